# baseline (speedup 1.0000x reference)
import functools

import jax
import jax.numpy as jnp
from jax import lax
from jax.experimental import pallas as pl
from jax.experimental.pallas import tpu as pltpu
from jax.experimental.pallas import tpu_sc as plsc

_ROWS = 16384
_SEQ = 200
_D = 128
_N = _ROWS * _SEQ
_NC = 2
_NS = 16
_NW = _NC * _NS
_PER_W = _N // _NW
_CH = 400
_NPAIR = _PER_W // (2 * _CH)

_mesh = plsc.VectorSubcoreMesh(core_axis_name="c", subcore_axis_name="s")


@functools.partial(
    pl.kernel,
    out_type=jax.ShapeDtypeStruct((_N, _D), jnp.float32),
    mesh=_mesh,
    scratch_types=[
        pltpu.VMEM((2, _D), jnp.float32),
        pltpu.VMEM((_CH,), jnp.int32),
        pltpu.VMEM((_CH,), jnp.int32),
        pltpu.VMEM((_CH, _D), jnp.float32),
        pltpu.VMEM((_CH, _D), jnp.float32),
        pltpu.SemaphoreType.DMA,
        pltpu.SemaphoreType.DMA,
        pltpu.SemaphoreType.DMA,
        pltpu.SemaphoreType.DMA,
    ],
)
def _sc_embed(pos_hbm, w_hbm, out_hbm, w_v, pos_v0, pos_v1, out_v0, out_v1,
              psem0, psem1, osem0, osem1):
    wid = lax.axis_index("s") * _NC + lax.axis_index("c")
    base = wid * _PER_W

    pltpu.sync_copy(w_hbm, w_v)
    w0 = [w_v[0, pl.ds(k * 16, 16)] for k in range(8)]
    df = [w_v[1, pl.ds(k * 16, 16)] - w0[k] for k in range(8)]

    def pos_copy(c, buf, sem):
        return pltpu.make_async_copy(
            pos_hbm.at[pl.ds(base + c * _CH, _CH)], buf, sem)

    def out_copy(c, buf, sem):
        return pltpu.make_async_copy(
            buf, out_hbm.at[pl.ds(base + c * _CH, _CH)], sem)

    def compute(pos_b, out_b):
        def grp(g, carry):
            jbase = g * 16
            pv = pos_b[pl.ds(jbase, 16)].astype(jnp.float32)
            for l in range(16):
                pf = pv[l]
                for k in range(8):
                    out_b[jbase + l, pl.ds(k * 16, 16)] = w0[k] + pf * df[k]
            return carry
        lax.fori_loop(0, _CH // 16, grp, 0)

    pos_copy(0, pos_v0, psem0).start()
    pos_copy(1, pos_v1, psem1).start()

    def pair(i, carry):
        c0 = 2 * i
        for b, (pos_v, out_v, psem, osem) in enumerate(
                ((pos_v0, out_v0, psem0, osem0), (pos_v1, out_v1, psem1, osem1))):
            c = c0 + b
            pos_copy(c, pos_v, psem).wait()

            @pl.when(i > 0)
            def _():
                out_copy(c - 2, out_v, osem).wait()

            compute(pos_v, out_v)
            out_copy(c, out_v, osem).start()

            @pl.when(i < _NPAIR - 1)
            def _():
                pos_copy(c + 2, pos_v, psem).start()
        return carry

    lax.fori_loop(0, _NPAIR, pair, 0)
    out_copy(2 * _NPAIR - 2, out_v0, osem0).wait()
    out_copy(2 * _NPAIR - 1, out_v1, osem1).wait()


def kernel(pos, seg_embd_weight):
    pos_flat = pos.astype(jnp.int32).reshape(_N)
    out = _sc_embed(pos_flat, seg_embd_weight)
    return out.reshape(_ROWS, _SEQ, _D)
